# baseline (speedup 1.0000x reference)
import functools

import jax
import jax.numpy as jnp
from jax import lax
from jax.experimental import pallas as pl
from jax.experimental.pallas import tpu as pltpu
from jax.experimental.pallas import tpu_sc as plsc

_NC, _NS = 2, 16
_NW = _NC * _NS
_B = 4096
_F = 4
_E = 64
_N_IDX = _B * _F
_PER = _N_IDX // _NW
_CH = 128
_NCH = _PER // _CH


def _sc_gather(xi, embed_table, lin16_table):
    mesh = plsc.VectorSubcoreMesh(
        core_axis_name="c", subcore_axis_name="s",
        num_cores=_NC, num_subcores=_NS)

    @functools.partial(
        pl.kernel,
        out_type=(
            jax.ShapeDtypeStruct((_N_IDX, _E), jnp.float32),
            jax.ShapeDtypeStruct((_NW, _PER), jnp.float32),
        ),
        mesh=mesh,
        scratch_types=[
            pltpu.VMEM((_NCH, _CH), jnp.int32),
            pltpu.VMEM((_NCH, _CH), jnp.int32),
            pltpu.VMEM((_PER, _E), jnp.float32),
            pltpu.VMEM((_PER, 16), jnp.float32),
            pltpu.VMEM((_PER,), jnp.float32),
            pltpu.SemaphoreType.DMA,
            pltpu.SemaphoreType.DMA,
        ],
        compiler_params=pltpu.CompilerParams(
            use_tc_tiling_on_sc=False, needs_layout_passes=False),
    )
    def k(xi_hbm, emb_hbm, lin_hbm, emb_out, lin_out, idx_v, idx16_v,
          rows_v, lin16_v, lv_v, sem_e, sem_l):
        wid = lax.axis_index("s") * _NC + lax.axis_index("c")
        base = wid * _PER
        pltpu.sync_copy(xi_hbm.at[wid], idx_v)
        for j in range(_NCH):
            for kk in range(_CH // 16):
                sl = pl.ds(kk * 16, 16)
                idx16_v[j, sl] = lax.shift_right_logical(idx_v[j, sl], 4)
        copies = []
        for j in range(_NCH):
            copies.append(pltpu.async_copy(
                emb_hbm.at[idx_v.at[j]],
                rows_v.at[pl.ds(j * _CH, _CH)], sem_e))
            copies.append(pltpu.async_copy(
                lin_hbm.at[idx16_v.at[j]],
                lin16_v.at[pl.ds(j * _CH, _CH)], sem_l))
        for c in copies:
            c.wait()
        lane16 = lax.iota(jnp.int32, 16)
        for g in range(_PER // 16):
            j, off = (g * 16) // _CH, (g * 16) % _CH
            lanes = lax.bitwise_and(idx_v[j, pl.ds(off, 16)], 15)
            rows = jnp.full((16,), g * 16, jnp.int32) + lane16
            lv_v[pl.ds(g * 16, 16)] = plsc.load_gather(
                lin16_v, [rows, lanes])
        pltpu.sync_copy(rows_v, emb_out.at[pl.ds(base, _PER)])
        pltpu.sync_copy(lv_v, lin_out.at[wid])

    return k(xi, embed_table, lin16_table)


def _bn_relu(h, g, be):
    mu = jnp.mean(h, axis=0, keepdims=True)
    d = h - mu
    var = jnp.mean(d * d, axis=0, keepdims=True)
    return jnp.maximum(g * d * lax.rsqrt(var + 1e-5) + be, 0.0)


def _dense_body(emb_ref, lin_ref, w1_ref, b1_ref, g1_ref, be1_ref,
                w2_ref, b2_ref, g2_ref, be2_ref, w3_ref, b3_ref,
                bias_ref, out_ref):
    e = emb_ref[...]
    e0 = e[:, 0 * _E:1 * _E]
    e1 = e[:, 1 * _E:2 * _E]
    e2 = e[:, 2 * _E:3 * _E]
    e3 = e[:, 3 * _E:4 * _E]
    s = e0 + e1 + e2 + e3
    sq = s * s - (e0 * e0 + e1 * e1 + e2 * e2 + e3 * e3)
    fm = 0.5 * jnp.sum(sq, axis=1, keepdims=True)
    lin = jnp.sum(lin_ref[...], axis=1, keepdims=True) + bias_ref[0, 0]
    h = jnp.dot(e, w1_ref[...], preferred_element_type=jnp.float32)
    h = _bn_relu(h + b1_ref[...], g1_ref[...], be1_ref[...])
    h = jnp.dot(h, w2_ref[...], preferred_element_type=jnp.float32)
    h = _bn_relu(h + b2_ref[...], g2_ref[...], be2_ref[...])
    z = jnp.dot(h, w3_ref[...], preferred_element_type=jnp.float32)
    z = z + b3_ref[0, 0] + lin + fm
    out_ref[...] = 1.0 / (1.0 + jnp.exp(-z))


def _dense(emb, lin4, W1, b1, g1, be1, W2, b2, g2, be2, W3, b3, lin_bias,
           interpret=False):
    return pl.pallas_call(
        _dense_body,
        out_shape=jax.ShapeDtypeStruct((_B, 1), jnp.float32),
        interpret=interpret,
    )(emb, lin4, W1, b1.reshape(1, -1), g1.reshape(1, -1), be1.reshape(1, -1),
      W2, b2.reshape(1, -1), g2.reshape(1, -1), be2.reshape(1, -1),
      W3, b3.reshape(1, 1), lin_bias.reshape(1, 1))


def kernel(x, embed_table, lin_table, lin_bias, W1, b1, g1, be1,
           W2, b2, g2, be2, W3, b3):
    offsets = jnp.arange(_F, dtype=jnp.int32) * 100000
    xi = (x.astype(jnp.int32) + offsets[None, :]).reshape(_NW, _NCH, _CH)
    lin16 = lin_table.reshape(25000, 16)
    emb_rows, lin_rows = _sc_gather(xi, embed_table, lin16)
    emb = emb_rows.reshape(_B, _F * _E)
    lin4 = lin_rows.reshape(_B, _F)
    return _dense(emb, lin4, W1, b1, g1, be1, W2, b2, g2, be2, W3, b3,
                  lin_bias)
